# Initial kernel scaffold; baseline (speedup 1.0000x reference)
import jax
import jax.numpy as jnp
from jax.experimental import pallas as pl


def kernel(node_features, edge_index, edges_weight, W0, b0, gamma0, beta0, W1, b1, gamma1, beta1, W2, b2, gamma2, beta2):
    raise NotImplementedError("write your pallas kernel here")


# speedup vs baseline: 10.4593x; 10.4593x over previous
import functools

import jax
import jax.numpy as jnp
from jax import lax
from jax.experimental import pallas as pl
from jax.experimental.pallas import tpu as pltpu
from jax.experimental.pallas import tpu_sc as plsc

NC = 2
NS = 16
L = 16
KB = 128


def _sc_deg(col, ew, n):
    e = col.shape[0]
    per_w = e // (NC * NS)
    nfull = per_w // KB
    rem = per_w - nfull * KB
    rows_per = (n // NS) // 8 * 8
    rows_tail = n - NS * rows_per

    mesh = plsc.VectorSubcoreMesh(core_axis_name="c", subcore_axis_name="s",
                                  num_cores=NC, num_subcores=NS)

    @functools.partial(
        pl.kernel,
        out_type=jax.ShapeDtypeStruct((NC, n, L), jnp.float32),
        mesh=mesh,
        compiler_params=pltpu.CompilerParams(use_tc_tiling_on_sc=False),
        scratch_types=[
            pltpu.VMEM_SHARED((n, L), jnp.float32),
            pltpu.VMEM((KB, L), jnp.float32),
            pltpu.VMEM((KB,), jnp.int32),
            pltpu.VMEM((KB,), jnp.float32),
            pltpu.VMEM((L, L), jnp.float32),
            pltpu.VMEM((L,), jnp.int32),
            pltpu.VMEM((L,), jnp.float32),
        ],
    )
    def k(col_hbm, ew_hbm, out_hbm, acc, msg, cbuf, ebuf, msg_t, cbuf_t, ebuf_t):
        c = lax.axis_index("c")
        s = lax.axis_index("s")
        zero = jnp.zeros((L,), jnp.float32)

        def zrow(kk, _):
            msg[kk, :] = zero
            return 0
        lax.fori_loop(0, KB, zrow, 0)

        base = s * rows_per
        nz = rows_per // KB
        for i in range(nz):
            pltpu.sync_copy(msg, acc.at[pl.ds(base + i * KB, KB)])
        tz = rows_per - nz * KB
        if tz:
            pltpu.sync_copy(msg.at[pl.ds(0, tz)], acc.at[pl.ds(base + nz * KB, tz)])
        if rows_tail:
            @pl.when(s == NS - 1)
            def _():
                pltpu.sync_copy(msg.at[pl.ds(0, rows_tail)],
                                acc.at[pl.ds(NS * rows_per, rows_tail)])
        plsc.subcore_barrier()

        wbase = (c * NS + s) * per_w

        def fill(nb, ebuf_, msg_):
            def body(g, _):
                ew16 = ebuf_[pl.ds(g * L, L)]
                for jj in range(L):
                    msg_[g * L + jj, :] = jnp.full((L,), ew16[jj], jnp.float32)
                return 0
            lax.fori_loop(0, nb // L, body, 0)

        def batch(ib, _):
            eb = wbase + ib * KB
            pltpu.sync_copy(col_hbm.at[pl.ds(eb, KB)], cbuf)
            pltpu.sync_copy(ew_hbm.at[pl.ds(eb, KB)], ebuf)
            fill(KB, ebuf, msg)
            pltpu.sync_copy(msg, acc.at[cbuf], add=True)
            return 0
        lax.fori_loop(0, nfull, batch, 0)

        if rem:
            eb = wbase + nfull * KB
            pltpu.sync_copy(col_hbm.at[pl.ds(eb, rem)], cbuf_t.at[pl.ds(0, rem)])
            pltpu.sync_copy(ew_hbm.at[pl.ds(eb, rem)], ebuf_t.at[pl.ds(0, rem)])
            fill(rem, ebuf_t, msg_t)
            pltpu.sync_copy(msg_t.at[pl.ds(0, rem)], acc.at[cbuf_t], add=True)

        plsc.subcore_barrier()
        pltpu.sync_copy(acc.at[pl.ds(base, rows_per)],
                        out_hbm.at[c, pl.ds(base, rows_per)])
        if rows_tail:
            @pl.when(s == NS - 1)
            def _():
                pltpu.sync_copy(acc.at[pl.ds(NS * rows_per, rows_tail)],
                                out_hbm.at[c, pl.ds(NS * rows_per, rows_tail)])

    return k(col, ew)


def _sc_aggregate(xwp, row, col, ew, n, d):
    e = row.shape[0]
    per_w = e // (NC * NS)
    nfull = per_w // KB
    rem = per_w - nfull * KB
    rows_per = (n // NS) // 8 * 8
    rows_tail = n - NS * rows_per
    nd = d // L

    mesh = plsc.VectorSubcoreMesh(core_axis_name="c", subcore_axis_name="s",
                                  num_cores=NC, num_subcores=NS)

    @functools.partial(
        pl.kernel,
        out_type=jax.ShapeDtypeStruct((NC, n, d), jnp.float32),
        mesh=mesh,
        scratch_types=[
            pltpu.VMEM_SHARED((n, d), jnp.float32),
            pltpu.VMEM((KB, d), jnp.float32),
            pltpu.VMEM((KB,), jnp.int32),
            pltpu.VMEM((KB,), jnp.int32),
            pltpu.VMEM((KB,), jnp.float32),
            pltpu.VMEM((L, d), jnp.float32),
            pltpu.VMEM((L,), jnp.int32),
            pltpu.VMEM((L,), jnp.int32),
            pltpu.VMEM((L,), jnp.float32),
            pltpu.SemaphoreType.DMA,
        ],
    )
    def k(xwp_hbm, row_hbm, col_hbm, ew_hbm, out_hbm,
          acc, msg, rbuf, cbuf, ebuf, msg_t, rbuf_t, cbuf_t, ebuf_t, sem):
        c = lax.axis_index("c")
        s = lax.axis_index("s")
        zero = jnp.zeros((L,), jnp.float32)

        def zrow(kk, _):
            for dd in range(nd):
                msg[kk, pl.ds(dd * L, L)] = zero
            return 0
        lax.fori_loop(0, KB, zrow, 0)

        base = s * rows_per
        nz = rows_per // KB
        for i in range(nz):
            pltpu.sync_copy(msg, acc.at[pl.ds(base + i * KB, KB)])
        tz = rows_per - nz * KB
        if tz:
            pltpu.sync_copy(msg.at[pl.ds(0, tz)], acc.at[pl.ds(base + nz * KB, tz)])
        if rows_tail:
            @pl.when(s == NS - 1)
            def _():
                pltpu.sync_copy(msg.at[pl.ds(0, rows_tail)],
                                acc.at[pl.ds(NS * rows_per, rows_tail)])
        plsc.subcore_barrier()

        wbase = (c * NS + s) * per_w

        def scale(nb, ebuf_, msg_):
            def body(g, _):
                ew16 = ebuf_[pl.ds(g * L, L)]
                for jj in range(L):
                    kk = g * L + jj
                    ewv = jnp.full((L,), ew16[jj], jnp.float32)
                    for dd in range(nd):
                        msg_[kk, pl.ds(dd * L, L)] = msg_[kk, pl.ds(dd * L, L)] * ewv
                return 0
            lax.fori_loop(0, nb // L, body, 0)

        def batch(ib, _):
            eb = wbase + ib * KB
            pltpu.sync_copy(row_hbm.at[pl.ds(eb, KB)], rbuf)
            pltpu.sync_copy(col_hbm.at[pl.ds(eb, KB)], cbuf)
            pltpu.sync_copy(ew_hbm.at[pl.ds(eb, KB)], ebuf)
            pltpu.async_copy(xwp_hbm.at[rbuf], msg, sem).wait()
            scale(KB, ebuf, msg)
            pltpu.sync_copy(msg, acc.at[cbuf], add=True)
            return 0
        lax.fori_loop(0, nfull, batch, 0)

        if rem:
            eb = wbase + nfull * KB
            pltpu.sync_copy(row_hbm.at[pl.ds(eb, rem)], rbuf_t.at[pl.ds(0, rem)])
            pltpu.sync_copy(col_hbm.at[pl.ds(eb, rem)], cbuf_t.at[pl.ds(0, rem)])
            pltpu.sync_copy(ew_hbm.at[pl.ds(eb, rem)], ebuf_t.at[pl.ds(0, rem)])
            pltpu.async_copy(xwp_hbm.at[rbuf_t], msg_t, sem).wait()
            scale(rem, ebuf_t, msg_t)
            pltpu.sync_copy(msg_t.at[pl.ds(0, rem)], acc.at[cbuf_t], add=True)

        plsc.subcore_barrier()
        pltpu.sync_copy(acc.at[pl.ds(base, rows_per)],
                        out_hbm.at[c, pl.ds(base, rows_per)])
        if rows_tail:
            @pl.when(s == NS - 1)
            def _():
                pltpu.sync_copy(acc.at[pl.ds(NS * rows_per, rows_tail)],
                                out_hbm.at[c, pl.ds(NS * rows_per, rows_tail)])

    return k(xwp, row, col, ew)


def _tc_pre(x, w0, degp, n, d):
    def body(x_ref, w_ref, degp_ref, dinv_ref, xwp_ref):
        dsum = jnp.sum(degp_ref[0], axis=1, keepdims=True) \
             + jnp.sum(degp_ref[1], axis=1, keepdims=True)
        deg = 1.0 + dsum * (1.0 / L)
        dinv = lax.rsqrt(deg)
        dinv_ref[...] = dinv
        xw = lax.dot_general(x_ref[...], w_ref[...], (((1,), (1,)), ((), ())),
                             preferred_element_type=jnp.float32)
        xwp_ref[...] = xw * dinv

    return pl.pallas_call(
        body,
        out_shape=(jax.ShapeDtypeStruct((n, 1), jnp.float32),
                   jax.ShapeDtypeStruct((n, d), jnp.float32)),
    )(x, w0, degp)


def _tc_post(sp, xwp, dinv, b, gamma, beta, w_next, n, d):
    has_next = w_next is not None

    def body(*refs):
        if has_next:
            sp_ref, xwp_ref, dinv_ref, b_ref, g_ref, be_ref, wn_ref, out_ref = refs
        else:
            sp_ref, xwp_ref, dinv_ref, b_ref, g_ref, be_ref, out_ref = refs
        dinv = dinv_ref[...]
        pre = dinv * (sp_ref[0] + sp_ref[1] + xwp_ref[...]) + b_ref[...]
        mean = jnp.mean(pre, axis=0, keepdims=True)
        var = jnp.mean((pre - mean) ** 2, axis=0, keepdims=True)
        h = (pre - mean) * lax.rsqrt(var + 1e-5) * g_ref[...] + be_ref[...]
        h = jnp.maximum(h, 0.0)
        if has_next:
            xw = lax.dot_general(h, wn_ref[...], (((1,), (1,)), ((), ())),
                                 preferred_element_type=jnp.float32)
            out_ref[...] = xw * dinv
        else:
            out_ref[...] = h

    args = [sp, xwp, dinv, b, gamma, beta]
    if has_next:
        args.append(w_next)
    return pl.pallas_call(
        body,
        out_shape=jax.ShapeDtypeStruct((n, d), jnp.float32),
    )(*args)


def kernel(node_features, edge_index, edges_weight,
           W0, b0, gamma0, beta0, W1, b1, gamma1, beta1, W2, b2, gamma2, beta2):
    n, d = node_features.shape
    row = edge_index[0]
    col = edge_index[1]
    ew = edges_weight

    degp = _sc_deg(col, ew, n)
    dinv, xwp = _tc_pre(node_features, W0, degp, n, d)

    params = [(b0, gamma0, beta0, W1), (b1, gamma1, beta1, W2),
              (b2, gamma2, beta2, None)]
    for b, g, be, w_next in params:
        sp = _sc_aggregate(xwp, row, col, ew, n, d)
        xwp = _tc_post(sp, xwp, dinv, b, g, be, w_next, n, d)
    return xwp
